# baseline (speedup 1.0000x reference)
import jax
import jax.numpy as jnp
from jax import lax
from jax.experimental import pallas as pl
from jax.experimental.pallas import tpu as pltpu
from jax.experimental.pallas import tpu_sc as plsc
from jax._src.pallas import mpmd

BATCH = 4
SEQ = 4096
DIM = 1024

_NUM_CORES = 2
_NUM_SUBCORES = 16
_NW = _NUM_CORES * _NUM_SUBCORES

_TEC_ROWS = 2048
_SCS_ROWS = SEQ - _TEC_ROWS

_T_CHUNK = 64
_T_PER_W = _TEC_ROWS // _NW
_T_NCHUNK = _T_PER_W // _T_CHUNK
_T_NBUF = 1

_S_CHUNK = 512
_S_PER_C = _SCS_ROWS // _NUM_CORES
_S_NCHUNK = _S_PER_C // _S_CHUNK
_S_NBUF = 2


def _pipeline(table_hbm, out_hbm, bufs, gsems, ssems, base, chunk, nchunk, nbuf):
    def gather(i):
        k = i % nbuf
        return pltpu.async_copy(
            table_hbm.at[pl.ds(base + i * chunk, chunk)], bufs[k], gsems[k]
        )

    def stores(i):
        k = i % nbuf
        return [
            pltpu.async_copy(
                bufs[k], out_hbm.at[b, pl.ds(base + i * chunk, chunk)], ssems[k]
            )
            for b in range(BATCH)
        ]

    g = {}
    st = {}
    waited = set()
    for i in range(min(nbuf, nchunk)):
        g[i] = gather(i)
    for i in range(nchunk):
        g[i].wait()
        st[i] = stores(i)
        j = i + nbuf
        if j < nchunk:
            for h in st[i]:
                h.wait()
            waited.add(i)
            g[j] = gather(j)
    for i in range(nchunk):
        if i not in waited:
            for h in st[i]:
                h.wait()


def _run(table):
    scalar_mesh = plsc.ScalarSubcoreMesh(axis_name="c", num_cores=_NUM_CORES)
    vector_mesh = plsc.VectorSubcoreMesh(
        core_axis_name="c", subcore_axis_name="s", num_cores=_NUM_CORES
    )

    vmem = pltpu.MemorySpace.VMEM @ vector_mesh
    spmem = pltpu.MemorySpace.VMEM_SHARED
    tec_sem = pltpu.SemaphoreType.DMA @ vector_mesh
    scs_sem = pltpu.SemaphoreType.DMA @ scalar_mesh

    scratch = (
        [vmem((_T_CHUNK, DIM), jnp.float32) for _ in range(_T_NBUF)]
        + [tec_sem for _ in range(2 * _T_NBUF)]
        + [spmem((_S_CHUNK, DIM), jnp.float32) for _ in range(_S_NBUF)]
        + [scs_sem for _ in range(2 * _S_NBUF)]
    )
    _N_T = 3 * _T_NBUF

    def tec_fn(table_hbm, out_hbm, *refs):
        bufs = refs[:_T_NBUF]
        gsems = refs[_T_NBUF : 2 * _T_NBUF]
        ssems = refs[2 * _T_NBUF : 3 * _T_NBUF]
        wid = lax.axis_index("s") * _NUM_CORES + lax.axis_index("c")
        base = wid * _T_PER_W
        _pipeline(
            table_hbm, out_hbm, bufs, gsems, ssems, base, _T_CHUNK, _T_NCHUNK, _T_NBUF
        )

    def scs_fn(table_hbm, out_hbm, *refs):
        bufs = refs[_N_T : _N_T + _S_NBUF]
        gsems = refs[_N_T + _S_NBUF : _N_T + 2 * _S_NBUF]
        ssems = refs[_N_T + 2 * _S_NBUF : _N_T + 3 * _S_NBUF]
        cid = lax.axis_index("c")
        base = _TEC_ROWS + cid * _S_PER_C
        _pipeline(
            table_hbm, out_hbm, bufs, gsems, ssems, base, _S_CHUNK, _S_NCHUNK, _S_NBUF
        )

    run = mpmd.mpmd_map(
        [(scalar_mesh, scs_fn), (vector_mesh, tec_fn)],
        out_types=jax.ShapeDtypeStruct((BATCH, SEQ, DIM), jnp.float32),
        scratch_types=scratch,
    )
    return run(table)


def kernel(x, table):
    del x
    return _run(table)


# speedup vs baseline: 1.0230x; 1.0230x over previous
import jax
import jax.numpy as jnp
from jax import lax
from jax.experimental import pallas as pl
from jax.experimental.pallas import tpu as pltpu
from jax.experimental.pallas import tpu_sc as plsc
from jax._src.pallas import mpmd

BATCH = 4
SEQ = 4096
DIM = 1024

_NUM_CORES = 2
_NUM_SUBCORES = 16
_NW = _NUM_CORES * _NUM_SUBCORES

_TEC_ROWS = 3072
_SCS_ROWS = SEQ - _TEC_ROWS

_T_CHUNK = 96
_T_PER_W = _TEC_ROWS // _NW
_T_NCHUNK = _T_PER_W // _T_CHUNK
_T_NBUF = 1

_S_CHUNK = 512
_S_PER_C = _SCS_ROWS // _NUM_CORES
_S_NCHUNK = _S_PER_C // _S_CHUNK
_S_NBUF = 1


def _pipeline(table_hbm, out_hbm, bufs, gsems, ssems, base, chunk, nchunk, nbuf):
    def gather(i):
        k = i % nbuf
        return pltpu.async_copy(
            table_hbm.at[pl.ds(base + i * chunk, chunk)], bufs[k], gsems[k]
        )

    def stores(i):
        k = i % nbuf
        return [
            pltpu.async_copy(
                bufs[k], out_hbm.at[b, pl.ds(base + i * chunk, chunk)], ssems[k]
            )
            for b in range(BATCH)
        ]

    g = {}
    st = {}
    waited = set()
    for i in range(min(nbuf, nchunk)):
        g[i] = gather(i)
    for i in range(nchunk):
        g[i].wait()
        st[i] = stores(i)
        j = i + nbuf
        if j < nchunk:
            for h in st[i]:
                h.wait()
            waited.add(i)
            g[j] = gather(j)
    for i in range(nchunk):
        if i not in waited:
            for h in st[i]:
                h.wait()


def _run(table):
    scalar_mesh = plsc.ScalarSubcoreMesh(axis_name="c", num_cores=_NUM_CORES)
    vector_mesh = plsc.VectorSubcoreMesh(
        core_axis_name="c", subcore_axis_name="s", num_cores=_NUM_CORES
    )

    vmem = pltpu.MemorySpace.VMEM @ vector_mesh
    spmem = pltpu.MemorySpace.VMEM_SHARED
    tec_sem = pltpu.SemaphoreType.DMA @ vector_mesh
    scs_sem = pltpu.SemaphoreType.DMA @ scalar_mesh

    scratch = (
        [vmem((_T_CHUNK, DIM), jnp.float32) for _ in range(_T_NBUF)]
        + [tec_sem for _ in range(2 * _T_NBUF)]
        + [spmem((_S_CHUNK, DIM), jnp.float32) for _ in range(_S_NBUF)]
        + [scs_sem for _ in range(2 * _S_NBUF)]
    )
    _N_T = 3 * _T_NBUF

    def tec_fn(table_hbm, out_hbm, *refs):
        bufs = refs[:_T_NBUF]
        gsems = refs[_T_NBUF : 2 * _T_NBUF]
        ssems = refs[2 * _T_NBUF : 3 * _T_NBUF]
        wid = lax.axis_index("s") * _NUM_CORES + lax.axis_index("c")
        base = wid * _T_PER_W
        _pipeline(
            table_hbm, out_hbm, bufs, gsems, ssems, base, _T_CHUNK, _T_NCHUNK, _T_NBUF
        )

    def scs_fn(table_hbm, out_hbm, *refs):
        bufs = refs[_N_T : _N_T + _S_NBUF]
        gsems = refs[_N_T + _S_NBUF : _N_T + 2 * _S_NBUF]
        ssems = refs[_N_T + 2 * _S_NBUF : _N_T + 3 * _S_NBUF]
        cid = lax.axis_index("c")
        base = _TEC_ROWS + cid * _S_PER_C
        _pipeline(
            table_hbm, out_hbm, bufs, gsems, ssems, base, _S_CHUNK, _S_NCHUNK, _S_NBUF
        )

    run = mpmd.mpmd_map(
        [(scalar_mesh, scs_fn), (vector_mesh, tec_fn)],
        out_types=jax.ShapeDtypeStruct((BATCH, SEQ, DIM), jnp.float32),
        scratch_types=scratch,
    )
    return run(table)


def kernel(x, table):
    del x
    return _run(table)
